# baseline (speedup 1.0000x reference)
import jax
import jax.numpy as jnp
from jax import lax
from jax.experimental import pallas as pl
from jax.experimental.pallas import tpu as pltpu
from jax.experimental.pallas import tpu_sc as plsc

_VMIN = -3.0
_VMAX = 3.0
_NW = 63
_NC = 192
_SIGMA = (_VMAX - _VMIN) / (_NW - 1)

_P = 128
_RLO = -8.0
_NTAB = 80 * _P
_SCALE = _P / _SIGMA
_OFFSET = (-_VMIN / _SIGMA - _RLO) * _P + 0.5

_ROWS = 2 * _NC
_NWORK = 32
_PPW = _ROWS // _NWORK

_CROWS = 56
_CPP = 224 // _CROWS
_NCHUNK = _PPW * _CPP
_NVEC = 224 // 16


def _table_body(w_ref, tab_ref):
    j = lax.broadcasted_iota(jnp.int32, (64, _NTAB), 0).astype(jnp.float32)
    m = lax.broadcasted_iota(jnp.int32, (64, _NTAB), 1).astype(jnp.float32)
    r = _RLO + m * (1.0 / _P)
    d = r - j
    phi = jnp.exp(-0.5 * d * d)
    phi = jnp.where(j <= float(_NW - 1), phi, 0.0)
    tab_ref[...] = jnp.dot(
        w_ref[...], phi, preferred_element_type=jnp.float32,
        precision=lax.Precision.HIGHEST)


def _build_table(W):
    w_pad = jnp.concatenate([W, jnp.zeros((_NC, 1), jnp.float32)], axis=1)
    return pl.pallas_call(
        _table_body,
        out_shape=jax.ShapeDtypeStruct((_NC, _NTAB), jnp.float32),
    )(w_pad)


def _sc_body(x_hbm, tab_hbm, out_hbm,
             tb0, tb1, xb0, xb1, ob0, ob1,
             st0, st1, sx0, sx1, so0, so1):
    wid = lax.axis_index("s") * 2 + lax.axis_index("c")
    plane0 = wid * _PPW
    tbufs, sts = (tb0, tb1), (st0, st1)
    xbufs, sxs = (xb0, xb1), (sx0, sx1)
    obufs, sos = (ob0, ob1), (so0, so1)

    def tab_slice(lp):
        c = lax.rem(plane0 + lp, _NC)
        return tab_hbm.at[pl.ds(pl.multiple_of(c * _NTAB, 8), _NTAB)]

    def x_slice(t):
        p = plane0 + lax.div(t, _CPP)
        r0 = lax.rem(t, _CPP) * _CROWS
        return x_hbm.at[p, pl.ds(r0, _CROWS), :]

    def out_slice(t):
        p = plane0 + lax.div(t, _CPP)
        r0 = lax.rem(t, _CPP) * _CROWS
        return out_hbm.at[p, pl.ds(r0, _CROWS), :]

    for pp in range(2):
        pltpu.async_copy(tab_slice(pp), tbufs[pp], sts[pp])
    for b in range(2):
        pltpu.async_copy(x_slice(b), xbufs[b], sxs[b])

    def outer(jp, carry):
        for pp in range(2):
            lp = jp * 2 + pp
            tb = tbufs[pp]
            pltpu.make_async_copy(tab_slice(lp), tb, sts[pp]).wait()
            for cc in range(_CPP):
                t = lp * _CPP + cc
                b = cc % 2
                pltpu.make_async_copy(x_slice(t), xbufs[b], sxs[b]).wait()

                @pl.when(t >= 2)
                def _wait_out():
                    pltpu.make_async_copy(
                        obufs[b], out_slice(t), sos[b]).wait()

                xb, ob = xbufs[b], obufs[b]

                @plsc.parallel_loop(0, _CROWS, step=1, unroll=1)
                def body(r):
                    for v in range(_NVEC):
                        xv = xb[r, pl.ds(v * 16, 16)]
                        tt = xv * _SCALE + _OFFSET
                        q = tt.astype(jnp.int32)
                        qu = jnp.minimum(plsc.bitcast(q, jnp.uint32),
                                         jnp.uint32(_NTAB - 1))
                        q = plsc.bitcast(qu, jnp.int32)
                        ob[r, pl.ds(v * 16, 16)] = plsc.load_gather(tb, [q])

                pltpu.async_copy(ob, out_slice(t), sos[b])

                @pl.when(t + 2 < _NCHUNK)
                def _prefetch():
                    pltpu.async_copy(x_slice(t + 2), xbufs[b], sxs[b])

            @pl.when(lp + 2 < _PPW)
            def _tab_prefetch():
                pltpu.async_copy(tab_slice(lp + 2), tbufs[pp], sts[pp])
        return carry

    lax.fori_loop(0, _PPW // 2, outer, 0)
    for b in range(2):
        pltpu.make_async_copy(obufs[b], out_slice(b), sos[b]).wait()


def kernel(x, W):
    tab = _build_table(W)
    x3 = x.reshape(_ROWS, 224, 224)
    tab_flat = tab.reshape(_NC * _NTAB)
    mesh = plsc.VectorSubcoreMesh(core_axis_name="c", subcore_axis_name="s")
    fn = pl.kernel(
        _sc_body,
        out_type=jax.ShapeDtypeStruct((_ROWS, 224, 224), jnp.float32),
        mesh=mesh,
        compiler_params=pltpu.CompilerParams(needs_layout_passes=False),
        scratch_types=[
            pltpu.VMEM((_NTAB,), jnp.float32),
            pltpu.VMEM((_NTAB,), jnp.float32),
            pltpu.VMEM((_CROWS, 224), jnp.float32),
            pltpu.VMEM((_CROWS, 224), jnp.float32),
            pltpu.VMEM((_CROWS, 224), jnp.float32),
            pltpu.VMEM((_CROWS, 224), jnp.float32),
            pltpu.SemaphoreType.DMA,
            pltpu.SemaphoreType.DMA,
            pltpu.SemaphoreType.DMA,
            pltpu.SemaphoreType.DMA,
            pltpu.SemaphoreType.DMA,
            pltpu.SemaphoreType.DMA,
        ],
    )
    out3 = fn(x3, tab_flat)
    return out3.reshape(x.shape)


# speedup vs baseline: 1.0201x; 1.0201x over previous
import jax
import jax.numpy as jnp
from jax import lax
from jax.experimental import pallas as pl
from jax.experimental.pallas import tpu as pltpu
from jax.experimental.pallas import tpu_sc as plsc

_VMIN = -3.0
_VMAX = 3.0
_NW = 63
_NC = 192
_SIGMA = (_VMAX - _VMIN) / (_NW - 1)

_P = 128
_RLO = -8.0
_NTAB = 80 * _P
_SCALE = _P / _SIGMA
_OFFSET = (-_VMIN / _SIGMA - _RLO) * _P + 0.5

_ROWS = 2 * _NC
_NWORK = 32
_PPW = _ROWS // _NWORK

_CROWS = 56
_CPP = 224 // _CROWS
_NCHUNK = _PPW * _CPP
_NVEC = 224 // 16


def _table_body(w_ref, tab_ref):
    j = lax.broadcasted_iota(jnp.int32, (64, _NTAB), 0).astype(jnp.float32)
    m = lax.broadcasted_iota(jnp.int32, (64, _NTAB), 1).astype(jnp.float32)
    r = _RLO + m * (1.0 / _P)
    d = r - j
    phi = jnp.exp(-0.5 * d * d)
    phi = jnp.where(j <= float(_NW - 1), phi, 0.0)
    tab_ref[...] = jnp.dot(
        w_ref[...], phi, preferred_element_type=jnp.float32,
        precision=lax.Precision.HIGHEST)


def _build_table(W):
    w_pad = jnp.concatenate([W, jnp.zeros((_NC, 1), jnp.float32)], axis=1)
    return pl.pallas_call(
        _table_body,
        out_shape=jax.ShapeDtypeStruct((_NC, _NTAB), jnp.float32),
    )(w_pad)


def _sc_body(x_hbm, tab_hbm, out_hbm,
             tb0, tb1, xb0, xb1, ob0, ob1,
             st0, st1, sx0, sx1, so0, so1):
    wid = lax.axis_index("s") * 2 + lax.axis_index("c")
    plane0 = wid * _PPW
    tbufs, sts = (tb0, tb1), (st0, st1)
    xbufs, sxs = (xb0, xb1), (sx0, sx1)
    obufs, sos = (ob0, ob1), (so0, so1)

    def tab_slice(lp):
        c = lax.rem(plane0 + lp, _NC)
        return tab_hbm.at[pl.ds(pl.multiple_of(c * _NTAB, 8), _NTAB)]

    def x_slice(t):
        p = plane0 + lax.div(t, _CPP)
        r0 = lax.rem(t, _CPP) * _CROWS
        return x_hbm.at[p, pl.ds(r0, _CROWS), :]

    def out_slice(t):
        p = plane0 + lax.div(t, _CPP)
        r0 = lax.rem(t, _CPP) * _CROWS
        return out_hbm.at[p, pl.ds(r0, _CROWS), :]

    for pp in range(2):
        pltpu.async_copy(tab_slice(pp), tbufs[pp], sts[pp])
    for b in range(2):
        pltpu.async_copy(x_slice(b), xbufs[b], sxs[b])

    def outer(jp, carry):
        for pp in range(2):
            lp = jp * 2 + pp
            tb = tbufs[pp]
            pltpu.make_async_copy(tab_slice(lp), tb, sts[pp]).wait()
            for cc in range(_CPP):
                t = lp * _CPP + cc
                b = cc % 2
                pltpu.make_async_copy(x_slice(t), xbufs[b], sxs[b]).wait()

                @pl.when(t >= 2)
                def _wait_out():
                    pltpu.make_async_copy(
                        obufs[b], out_slice(t), sos[b]).wait()

                xb, ob = xbufs[b], obufs[b]

                @plsc.parallel_loop(0, _CROWS, step=1, unroll=2)
                def body(r):
                    for v in range(_NVEC):
                        xv = xb[r, pl.ds(v * 16, 16)]
                        tt = xv * _SCALE + _OFFSET
                        tt = jnp.minimum(
                            jnp.maximum(tt, 0.0), float(_NTAB - 1))
                        q = tt.astype(jnp.int32)
                        ob[r, pl.ds(v * 16, 16)] = plsc.load_gather(tb, [q])

                pltpu.async_copy(ob, out_slice(t), sos[b])

                @pl.when(t + 2 < _NCHUNK)
                def _prefetch():
                    pltpu.async_copy(x_slice(t + 2), xbufs[b], sxs[b])

            @pl.when(lp + 2 < _PPW)
            def _tab_prefetch():
                pltpu.async_copy(tab_slice(lp + 2), tbufs[pp], sts[pp])
        return carry

    lax.fori_loop(0, _PPW // 2, outer, 0)
    for b in range(2):
        pltpu.make_async_copy(obufs[b], out_slice(b), sos[b]).wait()


def kernel(x, W):
    tab = _build_table(W)
    x3 = x.reshape(_ROWS, 224, 224)
    tab_flat = tab.reshape(_NC * _NTAB)
    mesh = plsc.VectorSubcoreMesh(core_axis_name="c", subcore_axis_name="s")
    fn = pl.kernel(
        _sc_body,
        out_type=jax.ShapeDtypeStruct((_ROWS, 224, 224), jnp.float32),
        mesh=mesh,
        compiler_params=pltpu.CompilerParams(needs_layout_passes=False),
        scratch_types=[
            pltpu.VMEM((_NTAB,), jnp.float32),
            pltpu.VMEM((_NTAB,), jnp.float32),
            pltpu.VMEM((_CROWS, 224), jnp.float32),
            pltpu.VMEM((_CROWS, 224), jnp.float32),
            pltpu.VMEM((_CROWS, 224), jnp.float32),
            pltpu.VMEM((_CROWS, 224), jnp.float32),
            pltpu.SemaphoreType.DMA,
            pltpu.SemaphoreType.DMA,
            pltpu.SemaphoreType.DMA,
            pltpu.SemaphoreType.DMA,
            pltpu.SemaphoreType.DMA,
            pltpu.SemaphoreType.DMA,
        ],
    )
    out3 = fn(x3, tab_flat)
    return out3.reshape(x.shape)


# speedup vs baseline: 1.0263x; 1.0061x over previous
import jax
import jax.numpy as jnp
from jax import lax
from jax.experimental import pallas as pl
from jax.experimental.pallas import tpu as pltpu
from jax.experimental.pallas import tpu_sc as plsc

_VMIN = -3.0
_VMAX = 3.0
_NW = 63
_NC = 192
_SIGMA = (_VMAX - _VMIN) / (_NW - 1)

_P = 128
_RLO = -8.0
_NTAB = 80 * _P
_SCALE = _P / _SIGMA
_OFFSET = (-_VMIN / _SIGMA - _RLO) * _P + 0.5

_ROWS = 2 * _NC
_NWORK = 32
_PPW = _ROWS // _NWORK

_CROWS = 56
_CPP = 224 // _CROWS
_NCHUNK = _PPW * _CPP
_NVEC = 224 // 16


def _table_body(w_ref, tab_ref):
    j = lax.broadcasted_iota(jnp.int32, (64, _NTAB), 0).astype(jnp.float32)
    m = lax.broadcasted_iota(jnp.int32, (64, _NTAB), 1).astype(jnp.float32)
    r = _RLO + m * (1.0 / _P)
    d = r - j
    phi = jnp.exp(-0.5 * d * d)
    phi = jnp.where(j <= float(_NW - 1), phi, 0.0)
    tab_ref[...] = jnp.dot(
        w_ref[...], phi, preferred_element_type=jnp.float32,
        precision=lax.Precision.HIGHEST)


def _build_table(W):
    w_pad = jnp.concatenate([W, jnp.zeros((_NC, 1), jnp.float32)], axis=1)
    return pl.pallas_call(
        _table_body,
        out_shape=jax.ShapeDtypeStruct((_NC, _NTAB), jnp.float32),
    )(w_pad)


def _sc_body(x_hbm, tab_hbm, out_hbm,
             tb0, tb1, xb0, xb1, ob0, ob1,
             st0, st1, sx0, sx1, so0, so1):
    wid = lax.axis_index("s") * 2 + lax.axis_index("c")
    plane0 = wid * _PPW
    tbufs, sts = (tb0, tb1), (st0, st1)
    xbufs, sxs = (xb0, xb1), (sx0, sx1)
    obufs, sos = (ob0, ob1), (so0, so1)

    def tab_slice(lp):
        c = lax.rem(plane0 + lp, _NC)
        return tab_hbm.at[pl.ds(pl.multiple_of(c * _NTAB, 8), _NTAB)]

    def x_slice(t):
        p = plane0 + lax.div(t, _CPP)
        r0 = lax.rem(t, _CPP) * _CROWS
        return x_hbm.at[p, pl.ds(r0, _CROWS), :]

    def out_slice(t):
        p = plane0 + lax.div(t, _CPP)
        r0 = lax.rem(t, _CPP) * _CROWS
        return out_hbm.at[p, pl.ds(r0, _CROWS), :]

    for pp in range(2):
        pltpu.async_copy(tab_slice(pp), tbufs[pp], sts[pp])
    for b in range(2):
        pltpu.async_copy(x_slice(b), xbufs[b], sxs[b])

    def outer(jp, carry):
        for pp in range(2):
            lp = jp * 2 + pp
            tb = tbufs[pp]
            pltpu.make_async_copy(tab_slice(lp), tb, sts[pp]).wait()
            for cc in range(_CPP):
                t = lp * _CPP + cc
                b = cc % 2
                pltpu.make_async_copy(x_slice(t), xbufs[b], sxs[b]).wait()

                @pl.when(t >= 2)
                def _wait_out():
                    pltpu.make_async_copy(
                        obufs[b], out_slice(t), sos[b]).wait()

                xb, ob = xbufs[b], obufs[b]

                @plsc.parallel_loop(0, _CROWS, step=1, unroll=1)
                def body(r):
                    for v in range(_NVEC):
                        xv = xb[r, pl.ds(v * 16, 16)]
                        tt = xv * _SCALE + _OFFSET
                        tt = jnp.minimum(
                            jnp.maximum(tt, 0.0), float(_NTAB - 1))
                        q = tt.astype(jnp.int32)
                        ob[r, pl.ds(v * 16, 16)] = plsc.load_gather(tb, [q])

                pltpu.async_copy(ob, out_slice(t), sos[b])

                @pl.when(t + 2 < _NCHUNK)
                def _prefetch():
                    pltpu.async_copy(x_slice(t + 2), xbufs[b], sxs[b])

            @pl.when(lp + 2 < _PPW)
            def _tab_prefetch():
                pltpu.async_copy(tab_slice(lp + 2), tbufs[pp], sts[pp])
        return carry

    lax.fori_loop(0, _PPW // 2, outer, 0)
    for b in range(2):
        pltpu.make_async_copy(obufs[b], out_slice(b), sos[b]).wait()


def kernel(x, W):
    tab = _build_table(W)
    x3 = x.reshape(_ROWS, 224, 224)
    tab_flat = tab.reshape(_NC * _NTAB)
    mesh = plsc.VectorSubcoreMesh(core_axis_name="c", subcore_axis_name="s")
    fn = pl.kernel(
        _sc_body,
        out_type=jax.ShapeDtypeStruct((_ROWS, 224, 224), jnp.float32),
        mesh=mesh,
        compiler_params=pltpu.CompilerParams(needs_layout_passes=False),
        scratch_types=[
            pltpu.VMEM((_NTAB,), jnp.float32),
            pltpu.VMEM((_NTAB,), jnp.float32),
            pltpu.VMEM((_CROWS, 224), jnp.float32),
            pltpu.VMEM((_CROWS, 224), jnp.float32),
            pltpu.VMEM((_CROWS, 224), jnp.float32),
            pltpu.VMEM((_CROWS, 224), jnp.float32),
            pltpu.SemaphoreType.DMA,
            pltpu.SemaphoreType.DMA,
            pltpu.SemaphoreType.DMA,
            pltpu.SemaphoreType.DMA,
            pltpu.SemaphoreType.DMA,
            pltpu.SemaphoreType.DMA,
        ],
    )
    out3 = fn(x3, tab_flat)
    return out3.reshape(x.shape)


# speedup vs baseline: 1.1546x; 1.1250x over previous
import jax
import jax.numpy as jnp
from jax import lax
from jax.experimental import pallas as pl
from jax.experimental.pallas import tpu as pltpu
from jax.experimental.pallas import tpu_sc as plsc

_VMIN = -3.0
_VMAX = 3.0
_NW = 63
_NC = 192
_SIGMA = (_VMAX - _VMIN) / (_NW - 1)

_P = 128
_RLO = -8.0
_NTAB = 80 * _P
_SCALE = _P / _SIGMA
_OFFSET = (-_VMIN / _SIGMA - _RLO) * _P + 0.5

_ROWS = 2 * _NC
_NWORK = 32
_PPW = _ROWS // _NWORK

_CROWS = 56
_CPP = 224 // _CROWS
_NCHUNK = _PPW * _CPP
_NVEC = 224 // 16


def _table_body(w_ref, tab_ref):
    j = lax.broadcasted_iota(jnp.int32, (64, _NTAB), 0).astype(jnp.float32)
    m = lax.broadcasted_iota(jnp.int32, (64, _NTAB), 1).astype(jnp.float32)
    r = _RLO + m * (1.0 / _P)
    d = r - j
    phi = jnp.exp(-0.5 * d * d)
    phi = jnp.where(j <= float(_NW - 1), phi, 0.0)
    f = jnp.dot(
        w_ref[...], phi, preferred_element_type=jnp.float32,
        precision=lax.Precision.HIGHEST)
    tab_ref[...] = f.reshape(_NC * _NTAB // 128, 128)


def _build_table(W):
    w_pad = jnp.concatenate([W, jnp.zeros((_NC, 1), jnp.float32)], axis=1)
    return pl.pallas_call(
        _table_body,
        out_shape=jax.ShapeDtypeStruct((_NC * _NTAB // 128, 128), jnp.float32),
    )(w_pad)


def _sc_body(x_hbm, tab_hbm, out_hbm,
             tb0, tb1, xb0, xb1, ob0, ob1,
             st0, st1, sx0, sx1, so0, so1):
    wid = lax.axis_index("s") * 2 + lax.axis_index("c")
    plane0 = wid * _PPW
    tbufs, sts = (tb0, tb1), (st0, st1)
    xbufs, sxs = (xb0, xb1), (sx0, sx1)
    obufs, sos = (ob0, ob1), (so0, so1)

    def tab_slice(lp):
        c = lax.rem(plane0 + lp, _NC)
        return tab_hbm.at[pl.ds(pl.multiple_of(c * _NTAB, 8), _NTAB)]

    def x_slice(t):
        p = plane0 + lax.div(t, _CPP)
        r0 = lax.rem(t, _CPP) * _CROWS
        return x_hbm.at[p, pl.ds(r0, _CROWS), :]

    def out_slice(t):
        p = plane0 + lax.div(t, _CPP)
        r0 = lax.rem(t, _CPP) * _CROWS
        return out_hbm.at[p, pl.ds(r0, _CROWS), :]

    for pp in range(2):
        pltpu.async_copy(tab_slice(pp), tbufs[pp], sts[pp])
    for b in range(2):
        pltpu.async_copy(x_slice(b), xbufs[b], sxs[b])

    def outer(jp, carry):
        for pp in range(2):
            lp = jp * 2 + pp
            tb = tbufs[pp]
            pltpu.make_async_copy(tab_slice(lp), tb, sts[pp]).wait()
            for cc in range(_CPP):
                t = lp * _CPP + cc
                b = cc % 2
                pltpu.make_async_copy(x_slice(t), xbufs[b], sxs[b]).wait()

                @pl.when(t >= 2)
                def _wait_out():
                    pltpu.make_async_copy(
                        obufs[b], out_slice(t), sos[b]).wait()

                xb, ob = xbufs[b], obufs[b]

                @plsc.parallel_loop(0, _CROWS, step=1, unroll=1)
                def body(r):
                    for v in range(_NVEC):
                        xv = xb[r, pl.ds(v * 16, 16)]
                        tt = xv * _SCALE + _OFFSET
                        tt = jnp.minimum(
                            jnp.maximum(tt, 0.0), float(_NTAB - 1))
                        q = tt.astype(jnp.int32)
                        ob[r, pl.ds(v * 16, 16)] = plsc.load_gather(tb, [q])

                pltpu.async_copy(ob, out_slice(t), sos[b])

                @pl.when(t + 2 < _NCHUNK)
                def _prefetch():
                    pltpu.async_copy(x_slice(t + 2), xbufs[b], sxs[b])

            @pl.when(lp + 2 < _PPW)
            def _tab_prefetch():
                pltpu.async_copy(tab_slice(lp + 2), tbufs[pp], sts[pp])
        return carry

    lax.fori_loop(0, _PPW // 2, outer, 0)
    for b in range(2):
        pltpu.make_async_copy(obufs[b], out_slice(b), sos[b]).wait()


def kernel(x, W):
    tab = _build_table(W)
    x3 = x.reshape(_ROWS, 224, 224)
    tab_flat = tab.reshape(_NC * _NTAB)
    mesh = plsc.VectorSubcoreMesh(core_axis_name="c", subcore_axis_name="s")
    fn = pl.kernel(
        _sc_body,
        out_type=jax.ShapeDtypeStruct((_ROWS, 224, 224), jnp.float32),
        mesh=mesh,
        compiler_params=pltpu.CompilerParams(needs_layout_passes=False),
        scratch_types=[
            pltpu.VMEM((_NTAB,), jnp.float32),
            pltpu.VMEM((_NTAB,), jnp.float32),
            pltpu.VMEM((_CROWS, 224), jnp.float32),
            pltpu.VMEM((_CROWS, 224), jnp.float32),
            pltpu.VMEM((_CROWS, 224), jnp.float32),
            pltpu.VMEM((_CROWS, 224), jnp.float32),
            pltpu.SemaphoreType.DMA,
            pltpu.SemaphoreType.DMA,
            pltpu.SemaphoreType.DMA,
            pltpu.SemaphoreType.DMA,
            pltpu.SemaphoreType.DMA,
            pltpu.SemaphoreType.DMA,
        ],
    )
    out3 = fn(x3, tab_flat)
    return out3.reshape(x.shape)


# speedup vs baseline: 1.2265x; 1.0622x over previous
import jax
import jax.numpy as jnp
from jax import lax
from jax.experimental import pallas as pl
from jax.experimental.pallas import tpu as pltpu
from jax.experimental.pallas import tpu_sc as plsc

_VMIN = -3.0
_VMAX = 3.0
_NW = 63
_NC = 192
_SIGMA = (_VMAX - _VMIN) / (_NW - 1)

_P = 64
_RLO = -8.0
_NTAB = 80 * _P
_SCALE = _P / _SIGMA
_OFFSET = (-_VMIN / _SIGMA - _RLO) * _P + 0.5

_ROWS = 2 * _NC
_NWORK = 32
_PPW = _ROWS // _NWORK

_CROWS = 56
_CPP = 224 // _CROWS
_NCHUNK = _PPW * _CPP
_NVEC = 224 // 16


def _table_body(w_ref, tab_ref):
    j = lax.broadcasted_iota(jnp.int32, (64, _NTAB), 0).astype(jnp.float32)
    m = lax.broadcasted_iota(jnp.int32, (64, _NTAB), 1).astype(jnp.float32)
    r = _RLO + m * (1.0 / _P)
    d = r - j
    phi = jnp.exp(-0.5 * d * d)
    phi = jnp.where(j <= float(_NW - 1), phi, 0.0)
    f = jnp.dot(
        w_ref[...], phi, preferred_element_type=jnp.float32,
        precision=lax.Precision.HIGHEST)
    tab_ref[...] = f.reshape(_NC * _NTAB // 128, 128)


def _build_table(W):
    w_pad = jnp.concatenate([W, jnp.zeros((_NC, 1), jnp.float32)], axis=1)
    return pl.pallas_call(
        _table_body,
        out_shape=jax.ShapeDtypeStruct((_NC * _NTAB // 128, 128), jnp.float32),
    )(w_pad)


def _sc_body(x_hbm, tab_hbm, out_hbm,
             tb0, tb1, xb0, xb1, ob0, ob1,
             st0, st1, sx0, sx1, so0, so1):
    wid = lax.axis_index("s") * 2 + lax.axis_index("c")
    plane0 = wid * _PPW
    tbufs, sts = (tb0, tb1), (st0, st1)
    xbufs, sxs = (xb0, xb1), (sx0, sx1)
    obufs, sos = (ob0, ob1), (so0, so1)

    def tab_slice(lp):
        c = lax.rem(plane0 + lp, _NC)
        return tab_hbm.at[pl.ds(pl.multiple_of(c * _NTAB, 8), _NTAB)]

    def x_slice(t):
        p = plane0 + lax.div(t, _CPP)
        r0 = lax.rem(t, _CPP) * _CROWS
        return x_hbm.at[p, pl.ds(r0, _CROWS), :]

    def out_slice(t):
        p = plane0 + lax.div(t, _CPP)
        r0 = lax.rem(t, _CPP) * _CROWS
        return out_hbm.at[p, pl.ds(r0, _CROWS), :]

    for pp in range(2):
        pltpu.async_copy(tab_slice(pp), tbufs[pp], sts[pp])
    for b in range(2):
        pltpu.async_copy(x_slice(b), xbufs[b], sxs[b])

    def outer(jp, carry):
        for pp in range(2):
            lp = jp * 2 + pp
            tb = tbufs[pp]
            pltpu.make_async_copy(tab_slice(lp), tb, sts[pp]).wait()
            for cc in range(_CPP):
                t = lp * _CPP + cc
                b = cc % 2
                pltpu.make_async_copy(x_slice(t), xbufs[b], sxs[b]).wait()

                @pl.when(t >= 2)
                def _wait_out():
                    pltpu.make_async_copy(
                        obufs[b], out_slice(t), sos[b]).wait()

                xb, ob = xbufs[b], obufs[b]

                @plsc.parallel_loop(0, _CROWS, step=1, unroll=1)
                def body(r):
                    for v in range(_NVEC):
                        xv = xb[r, pl.ds(v * 16, 16)]
                        tt = xv * _SCALE + _OFFSET
                        tt = jnp.minimum(
                            jnp.maximum(tt, 0.0), float(_NTAB - 1))
                        q = tt.astype(jnp.int32)
                        ob[r, pl.ds(v * 16, 16)] = plsc.load_gather(tb, [q])

                pltpu.async_copy(ob, out_slice(t), sos[b])

                @pl.when(t + 2 < _NCHUNK)
                def _prefetch():
                    pltpu.async_copy(x_slice(t + 2), xbufs[b], sxs[b])

            @pl.when(lp + 2 < _PPW)
            def _tab_prefetch():
                pltpu.async_copy(tab_slice(lp + 2), tbufs[pp], sts[pp])
        return carry

    lax.fori_loop(0, _PPW // 2, outer, 0)
    for b in range(2):
        pltpu.make_async_copy(obufs[b], out_slice(b), sos[b]).wait()


def kernel(x, W):
    tab = _build_table(W)
    x3 = x.reshape(_ROWS, 224, 224)
    tab_flat = tab.reshape(_NC * _NTAB)
    mesh = plsc.VectorSubcoreMesh(core_axis_name="c", subcore_axis_name="s")
    fn = pl.kernel(
        _sc_body,
        out_type=jax.ShapeDtypeStruct((_ROWS, 224, 224), jnp.float32),
        mesh=mesh,
        compiler_params=pltpu.CompilerParams(needs_layout_passes=False),
        scratch_types=[
            pltpu.VMEM((_NTAB,), jnp.float32),
            pltpu.VMEM((_NTAB,), jnp.float32),
            pltpu.VMEM((_CROWS, 224), jnp.float32),
            pltpu.VMEM((_CROWS, 224), jnp.float32),
            pltpu.VMEM((_CROWS, 224), jnp.float32),
            pltpu.VMEM((_CROWS, 224), jnp.float32),
            pltpu.SemaphoreType.DMA,
            pltpu.SemaphoreType.DMA,
            pltpu.SemaphoreType.DMA,
            pltpu.SemaphoreType.DMA,
            pltpu.SemaphoreType.DMA,
            pltpu.SemaphoreType.DMA,
        ],
    )
    out3 = fn(x3, tab_flat)
    return out3.reshape(x.shape)


# speedup vs baseline: 1.2584x; 1.0260x over previous
import jax
import jax.numpy as jnp
from jax import lax
from jax.experimental import pallas as pl
from jax.experimental.pallas import tpu as pltpu
from jax.experimental.pallas import tpu_sc as plsc

_VMIN = -3.0
_VMAX = 3.0
_NW = 63
_NC = 192
_SIGMA = (_VMAX - _VMIN) / (_NW - 1)

_P = 32
_RLO = -8.0
_NTAB = 80 * _P
_SCALE = _P / _SIGMA
_OFFSET = (-_VMIN / _SIGMA - _RLO) * _P + 0.5

_ROWS = 2 * _NC
_NWORK = 32
_PPW = _ROWS // _NWORK

_CROWS = 56
_CPP = 224 // _CROWS
_NCHUNK = _PPW * _CPP
_NVEC = 224 // 16


def _table_body(w_ref, tab_ref):
    j = lax.broadcasted_iota(jnp.int32, (64, _NTAB), 0).astype(jnp.float32)
    m = lax.broadcasted_iota(jnp.int32, (64, _NTAB), 1).astype(jnp.float32)
    r = _RLO + m * (1.0 / _P)
    d = r - j
    phi = jnp.exp(-0.5 * d * d)
    phi = jnp.where(j <= float(_NW - 1), phi, 0.0)
    f = jnp.dot(
        w_ref[...], phi, preferred_element_type=jnp.float32,
        precision=lax.Precision.HIGHEST)
    tab_ref[...] = f.reshape(_NC * _NTAB // 128, 128)


def _build_table(W):
    w_pad = jnp.concatenate([W, jnp.zeros((_NC, 1), jnp.float32)], axis=1)
    return pl.pallas_call(
        _table_body,
        out_shape=jax.ShapeDtypeStruct((_NC * _NTAB // 128, 128), jnp.float32),
    )(w_pad)


def _sc_body(x_hbm, tab_hbm, out_hbm,
             tb0, tb1, xb0, xb1, ob0, ob1,
             st0, st1, sx0, sx1, so0, so1):
    wid = lax.axis_index("s") * 2 + lax.axis_index("c")
    plane0 = wid * _PPW
    tbufs, sts = (tb0, tb1), (st0, st1)
    xbufs, sxs = (xb0, xb1), (sx0, sx1)
    obufs, sos = (ob0, ob1), (so0, so1)

    def tab_slice(lp):
        c = lax.rem(plane0 + lp, _NC)
        return tab_hbm.at[pl.ds(pl.multiple_of(c * _NTAB, 8), _NTAB)]

    def x_slice(t):
        p = plane0 + lax.div(t, _CPP)
        r0 = lax.rem(t, _CPP) * _CROWS
        return x_hbm.at[p, pl.ds(r0, _CROWS), :]

    def out_slice(t):
        p = plane0 + lax.div(t, _CPP)
        r0 = lax.rem(t, _CPP) * _CROWS
        return out_hbm.at[p, pl.ds(r0, _CROWS), :]

    for pp in range(2):
        pltpu.async_copy(tab_slice(pp), tbufs[pp], sts[pp])
    for b in range(2):
        pltpu.async_copy(x_slice(b), xbufs[b], sxs[b])

    def outer(jp, carry):
        for pp in range(2):
            lp = jp * 2 + pp
            tb = tbufs[pp]
            pltpu.make_async_copy(tab_slice(lp), tb, sts[pp]).wait()
            for cc in range(_CPP):
                t = lp * _CPP + cc
                b = cc % 2
                pltpu.make_async_copy(x_slice(t), xbufs[b], sxs[b]).wait()

                @pl.when(t >= 2)
                def _wait_out():
                    pltpu.make_async_copy(
                        obufs[b], out_slice(t), sos[b]).wait()

                xb, ob = xbufs[b], obufs[b]

                @plsc.parallel_loop(0, _CROWS, step=1, unroll=1)
                def body(r):
                    for v in range(_NVEC):
                        xv = xb[r, pl.ds(v * 16, 16)]
                        tt = xv * _SCALE + _OFFSET
                        tt = jnp.minimum(
                            jnp.maximum(tt, 0.0), float(_NTAB - 1))
                        q = tt.astype(jnp.int32)
                        ob[r, pl.ds(v * 16, 16)] = plsc.load_gather(tb, [q])

                pltpu.async_copy(ob, out_slice(t), sos[b])

                @pl.when(t + 2 < _NCHUNK)
                def _prefetch():
                    pltpu.async_copy(x_slice(t + 2), xbufs[b], sxs[b])

            @pl.when(lp + 2 < _PPW)
            def _tab_prefetch():
                pltpu.async_copy(tab_slice(lp + 2), tbufs[pp], sts[pp])
        return carry

    lax.fori_loop(0, _PPW // 2, outer, 0)
    for b in range(2):
        pltpu.make_async_copy(obufs[b], out_slice(b), sos[b]).wait()


def kernel(x, W):
    tab = _build_table(W)
    x3 = x.reshape(_ROWS, 224, 224)
    tab_flat = tab.reshape(_NC * _NTAB)
    mesh = plsc.VectorSubcoreMesh(core_axis_name="c", subcore_axis_name="s")
    fn = pl.kernel(
        _sc_body,
        out_type=jax.ShapeDtypeStruct((_ROWS, 224, 224), jnp.float32),
        mesh=mesh,
        compiler_params=pltpu.CompilerParams(needs_layout_passes=False),
        scratch_types=[
            pltpu.VMEM((_NTAB,), jnp.float32),
            pltpu.VMEM((_NTAB,), jnp.float32),
            pltpu.VMEM((_CROWS, 224), jnp.float32),
            pltpu.VMEM((_CROWS, 224), jnp.float32),
            pltpu.VMEM((_CROWS, 224), jnp.float32),
            pltpu.VMEM((_CROWS, 224), jnp.float32),
            pltpu.SemaphoreType.DMA,
            pltpu.SemaphoreType.DMA,
            pltpu.SemaphoreType.DMA,
            pltpu.SemaphoreType.DMA,
            pltpu.SemaphoreType.DMA,
            pltpu.SemaphoreType.DMA,
        ],
    )
    out3 = fn(x3, tab_flat)
    return out3.reshape(x.shape)
